# baseline (speedup 1.0000x reference)
import functools

import jax
import jax.numpy as jnp
from jax import lax
from jax.experimental import pallas as pl
from jax.experimental.pallas import tpu as pltpu
from jax.experimental.pallas import tpu_sc as plsc

_MAX_LEN = 4194304
_N_APPENDS = 6291456
_H = _N_APPENDS - _MAX_LEN
_NW = 32
_PER_W = _MAX_LEN // _NW


_CHUNK = 16384
_NCHUNK = _PER_W // _CHUNK


@functools.partial(
    pl.kernel,
    mesh=plsc.VectorSubcoreMesh(core_axis_name="c", subcore_axis_name="s"),
    out_type=jax.ShapeDtypeStruct((_MAX_LEN,), jnp.float32),
    scratch_types=[pltpu.SemaphoreType.DMA],
)
def _ring_rotate(values_hbm, out_hbm, sem):
    wid = lax.axis_index("s") * 2 + lax.axis_index("c")
    dst = wid * _PER_W
    src = dst + jnp.where(dst < _H, _MAX_LEN, 0)
    cps = []
    for k in range(_NCHUNK):
        cp = pltpu.make_async_copy(
            values_hbm.at[pl.ds(src + k * _CHUNK, _CHUNK)],
            out_hbm.at[pl.ds(dst + k * _CHUNK, _CHUNK)],
            sem,
        )
        cp.start()
        cps.append(cp)
    for cp in cps:
        cp.wait()


def kernel(values, buffer):
    del buffer
    return _ring_rotate(values)


# speedup vs baseline: 16.4848x; 16.4848x over previous
import functools

import jax
import jax.numpy as jnp
from jax import lax
from jax.experimental import pallas as pl
from jax.experimental.pallas import tpu as pltpu
from jax.experimental.pallas import tpu_sc as plsc

_MAX_LEN = 4194304
_N_APPENDS = 6291456
_H = _N_APPENDS - _MAX_LEN
_NW = 32
_PER_W = _MAX_LEN // _NW


_CHUNK = 32768
_NCHUNK = _PER_W // _CHUNK


@functools.partial(
    pl.kernel,
    mesh=plsc.VectorSubcoreMesh(core_axis_name="c", subcore_axis_name="s"),
    out_type=jax.ShapeDtypeStruct((_MAX_LEN,), jnp.float32),
    scratch_types=[
        pltpu.VMEM((2, _CHUNK), jnp.float32),
        pltpu.SemaphoreType.DMA,
        pltpu.SemaphoreType.DMA,
    ],
)
def _ring_rotate(values_hbm, out_hbm, buf, sem_in, sem_out):
    wid = lax.axis_index("s") * 2 + lax.axis_index("c")
    dst = wid * _PER_W
    src = dst + jnp.where(dst < _H, _MAX_LEN, 0)

    def rd(k):
        return pltpu.make_async_copy(
            values_hbm.at[pl.ds(src + k * _CHUNK, _CHUNK)], buf.at[k % 2], sem_in)

    def wr(k):
        return pltpu.make_async_copy(
            buf.at[k % 2], out_hbm.at[pl.ds(dst + k * _CHUNK, _CHUNK)], sem_out)

    rd(0).start()
    for k in range(_NCHUNK):
        rd(k).wait()
        if k + 1 < _NCHUNK:
            if k >= 1:
                wr(k - 1).wait()
            rd(k + 1).start()
        wr(k).start()
    wr(_NCHUNK - 2).wait()
    wr(_NCHUNK - 1).wait()


def kernel(values, buffer):
    del buffer
    return _ring_rotate(values)
